# baseline (speedup 1.0000x reference)
import functools

import jax
import jax.numpy as jnp
from jax import lax
from jax.experimental import pallas as pl
from jax.experimental.pallas import tpu as pltpu
from jax.experimental.pallas import tpu_sc as plsc

VOCAB = 1000000
DIM = 32
B = 16384
L = 50

NC = 2
NS = 16
NW = NC * NS

TCOLS = (VOCAB + 127) // 128
VPAD = TCOLS * 128
WORDS = VPAD * (DIM // 2)
GCOLS = 4
NGRP = (TCOLS - 1) // GCOLS
GPW = (NGRP + NW - 1) // NW
GW = GCOLS * 128 * 16

SPW = B // NW
C = 16
ROWS = C * L
NCHUNK = SPW // C
IW = 100
NG = ROWS // IW
IDX_ROWS_TOTAL = B * L // IW

_mesh1 = plsc.VectorSubcoreMesh(core_axis_name="c", subcore_axis_name="s")
_mesh2 = plsc.VectorSubcoreMesh(core_axis_name="c", subcore_axis_name="s")


@functools.partial(
    pl.kernel,
    out_type=jax.ShapeDtypeStruct((WORDS,), jnp.float32),
    mesh=_mesh1,
    scratch_types=[
        pltpu.VMEM((4, 8, GCOLS * 128), jnp.float32),
        pltpu.VMEM((16 * 513 + 16,), jnp.float32),
        pltpu.VMEM((2 * GW,), jnp.float32),
        pltpu.SemaphoreType.DMA,
        pltpu.SemaphoreType.DMA,
    ],
    compiler_params=pltpu.CompilerParams(
        use_tc_tiling_on_sc=True,
        needs_layout_passes=False,
        disable_bounds_checks=True,
    ),
)
def _pack_kernel(tt_hbm, out_hbm, ibuf, abuf, obuf, sem_in, sem_out):
    wid = lax.axis_index("s") * NC + lax.axis_index("c")
    lane = lax.iota(jnp.int32, 16)
    iota513 = lane * 513

    def pack_cols(ncols, poff):
        for cc in range(ncols):
            for rc in range(8):
                rl0 = cc * 128 + rc * 16
                for j in range(16):
                    db = j // 4
                    ds = 2 * (j % 4)
                    e = ibuf[db, ds, pl.ds(rl0, 16)]
                    o = ibuf[db, ds + 1, pl.ds(rl0, 16)]
                    w = plsc.bitcast(
                        plsc.pack(e, o, format=plsc.PackFormat.INTERLEAVED),
                        jnp.float32,
                    )
                    abuf[pl.ds(j * 513 + rl0, 16)] = w
        for rg in range(ncols * 128):
            w16 = plsc.load_gather(abuf, [iota513 + rg])
            obuf[pl.ds(poff + rg * 16, 16)] = w16

    def grp_body(i, _):
        g = wid * GPW + i

        @pl.when(g < NGRP)
        def _():
            poff = lax.rem(i, 2) * GW

            @pl.when(i >= 2)
            def _():
                pltpu.make_async_copy(
                    obuf.at[pl.ds(0, GW)], out_hbm.at[pl.ds(0, GW)], sem_out
                ).wait()

            cps = [
                pltpu.async_copy(
                    tt_hbm.at[
                        pl.ds(8 * db, 8), pl.ds(g * (GCOLS * 128), GCOLS * 128)
                    ],
                    ibuf.at[db],
                    sem_in,
                )
                for db in range(4)
            ]
            for cp in cps:
                cp.wait()
            pack_cols(GCOLS, poff)
            pltpu.async_copy(
                obuf.at[pl.ds(poff, GW)], out_hbm.at[pl.ds(g * GW, GW)], sem_out
            )

        return 0

    lax.fori_loop(0, GPW, grp_body, 0)

    ngrp_w = jnp.minimum(jnp.maximum(NGRP - wid * GPW, 0), GPW)
    @pl.when(ngrp_w >= 1)
    def _():
        pltpu.make_async_copy(
            obuf.at[pl.ds(0, GW)], out_hbm.at[pl.ds(0, GW)], sem_out
        ).wait()

    @pl.when(ngrp_w >= 2)
    def _():
        pltpu.make_async_copy(
            obuf.at[pl.ds(0, GW)], out_hbm.at[pl.ds(0, GW)], sem_out
        ).wait()

    @pl.when(wid == 0)
    def _():
        c = NGRP * GCOLS + wid
        cps = [
            pltpu.async_copy(
                tt_hbm.at[pl.ds(8 * db, 8), pl.ds(c * 128, 128)],
                ibuf.at[db, :, pl.ds(0, 128)],
                sem_in,
            )
            for db in range(4)
        ]
        for cp in cps:
            cp.wait()
        pack_cols(1, 0)
        pltpu.sync_copy(
            obuf.at[pl.ds(0, 2048)], out_hbm.at[pl.ds(c * 2048, 2048)]
        )


@functools.partial(
    pl.kernel,
    out_type=jax.ShapeDtypeStruct((B, DIM), jnp.float32),
    mesh=_mesh2,
    scratch_types=[
        pltpu.VMEM((NG, IW), jnp.int32),
        pltpu.VMEM((ROWS, DIM // 2), jnp.float32),
        pltpu.VMEM((C, DIM), jnp.float32),
        pltpu.SemaphoreType.DMA,
    ],
    compiler_params=pltpu.CompilerParams(
        use_tc_tiling_on_sc=False, needs_layout_passes=False
    ),
)
def _embed_kernel(tw_hbm, batch_hbm, out_hbm, idx_v, rows_v, out_v, sem):
    wid = lax.axis_index("s") * NC + lax.axis_index("c")
    lane = lax.iota(jnp.int32, 16)
    even = lane * 2
    odd = even + 1

    def chunk_body(ci, _):
        idx_row0 = wid * (SPW * L // IW) + ci * NG
        pltpu.sync_copy(batch_hbm.at[pl.ds(idx_row0, NG)], idx_v)
        copies = [
            pltpu.async_copy(
                tw_hbm.at[idx_v.at[j]],
                rows_v.at[pl.ds(j * IW, IW)],
                sem,
            )
            for j in range(NG)
        ]
        for cp in copies:
            cp.wait()

        def acc_body(l, accs):
            out = []
            for s in range(C):
                ae, ao = accs[s]
                row = plsc.bitcast(rows_v[s * L + l, :], jnp.bfloat16)
                e, o = plsc.unpack(row, format=plsc.PackFormat.INTERLEAVED)
                out.append((ae + e, ao + o))
            return tuple(out)

        zero = jnp.zeros((16,), jnp.float32)
        init = tuple((zero, zero) for _ in range(C))
        accs = lax.fori_loop(0, L, acc_body, init)
        scale = jnp.float32(1.0 / L)
        for s in range(C):
            ae, ao = accs[s]
            srow = jnp.full((16,), s, jnp.int32)
            plsc.store_scatter(out_v, [srow, even], ae * scale)
            plsc.store_scatter(out_v, [srow, odd], ao * scale)

        base = wid * SPW + ci * C
        pltpu.sync_copy(out_v, out_hbm.at[pl.ds(base, C)])
        return 0

    lax.fori_loop(0, NCHUNK, chunk_body, 0)


def kernel(table, batch):
    words = _pack_kernel(table.T)
    tw = words.reshape(VPAD, DIM // 2)
    batch_r = batch.reshape(IDX_ROWS_TOTAL, IW)
    return _embed_kernel(tw, batch_r)


# speedup vs baseline: 1.4462x; 1.4462x over previous
import functools

import jax
import jax.numpy as jnp
from jax import lax
from jax.experimental import pallas as pl
from jax.experimental.pallas import tpu as pltpu
from jax.experimental.pallas import tpu_sc as plsc

VOCAB = 1000000
DIM = 32
B = 16384
L = 50

NC = 2
NS = 16
NW = NC * NS

TCOLS = (VOCAB + 127) // 128
VPAD = TCOLS * 128
WORDS = VPAD * (DIM // 2)
GCOLS = 4
NGRP = (TCOLS - 1) // GCOLS
GPW = (NGRP + NW - 1) // NW
GW = GCOLS * 128 * 16

SPW = B // NW
C = 16
ROWS = C * L
NCHUNK = SPW // C
IW = 100
NG = ROWS // IW
IDX_ROWS_TOTAL = B * L // IW

_mesh1 = plsc.VectorSubcoreMesh(core_axis_name="c", subcore_axis_name="s")
_mesh2 = plsc.VectorSubcoreMesh(core_axis_name="c", subcore_axis_name="s")


@functools.partial(
    pl.kernel,
    out_type=jax.ShapeDtypeStruct((WORDS,), jnp.float32),
    mesh=_mesh1,
    scratch_types=[
        pltpu.VMEM((2, 4, 8, GCOLS * 128), jnp.float32),
        pltpu.VMEM((2 * GW,), jnp.float32),
        pltpu.SemaphoreType.DMA,
        pltpu.SemaphoreType.DMA,
    ],
    compiler_params=pltpu.CompilerParams(
        use_tc_tiling_on_sc=True,
        needs_layout_passes=False,
        disable_bounds_checks=True,
    ),
)
def _pack_kernel(tt_hbm, out_hbm, ibuf, obuf, sem_in, sem_out):
    wid = lax.axis_index("s") * NC + lax.axis_index("c")
    lane16 = lax.iota(jnp.int32, 16) * 16
    g0 = wid * GPW
    ngrp_w = jnp.minimum(jnp.maximum(NGRP - g0, 0), GPW)

    def issue_in(g, pio):
        for db in range(4):
            pltpu.async_copy(
                tt_hbm.at[
                    pl.ds(8 * db, 8), pl.ds(g * (GCOLS * 128), GCOLS * 128)
                ],
                ibuf.at[pio, db],
                sem_in,
            )

    def pack_cols(pio, ncols, poff):
        for cc in range(ncols):
            for rc in range(8):
                rl0 = cc * 128 + rc * 16
                idx_base = lane16 + (rl0 * 16) + poff
                for j in range(16):
                    db = j // 4
                    ds = 2 * (j % 4)
                    e = ibuf[pio, db, ds, pl.ds(rl0, 16)]
                    o = ibuf[pio, db, ds + 1, pl.ds(rl0, 16)]
                    w = plsc.bitcast(
                        plsc.pack(e, o, format=plsc.PackFormat.INTERLEAVED),
                        jnp.float32,
                    )
                    plsc.store_scatter(obuf, [idx_base + j], w)

    @pl.when(ngrp_w >= 1)
    def _():
        issue_in(g0, 0)

    def grp_body(i, _):
        g = g0 + i

        @pl.when(g < NGRP)
        def _():
            pio = lax.rem(i, 2)
            poff = pio * GW
            for db in range(4):
                pltpu.make_async_copy(
                    tt_hbm.at[pl.ds(0, 8), pl.ds(0, GCOLS * 128)],
                    ibuf.at[0, db],
                    sem_in,
                ).wait()
            @pl.when(g + 1 < jnp.minimum(g0 + GPW, NGRP))
            def _():
                issue_in(g + 1, 1 - pio)

            @pl.when(i >= 2)
            def _():
                pltpu.make_async_copy(
                    obuf.at[pl.ds(0, GW)], out_hbm.at[pl.ds(0, GW)], sem_out
                ).wait()

            pack_cols(pio, GCOLS, poff)
            pltpu.async_copy(
                obuf.at[pl.ds(poff, GW)], out_hbm.at[pl.ds(g * GW, GW)], sem_out
            )

        return 0

    lax.fori_loop(0, GPW, grp_body, 0)

    @pl.when(ngrp_w >= 1)
    def _():
        pltpu.make_async_copy(
            obuf.at[pl.ds(0, GW)], out_hbm.at[pl.ds(0, GW)], sem_out
        ).wait()

    @pl.when(ngrp_w >= 2)
    def _():
        pltpu.make_async_copy(
            obuf.at[pl.ds(0, GW)], out_hbm.at[pl.ds(0, GW)], sem_out
        ).wait()

    @pl.when(wid == 0)
    def _():
        c = NGRP * GCOLS + wid
        cps = [
            pltpu.async_copy(
                tt_hbm.at[pl.ds(8 * db, 8), pl.ds(c * 128, 128)],
                ibuf.at[0, db, :, pl.ds(0, 128)],
                sem_in,
            )
            for db in range(4)
        ]
        for cp in cps:
            cp.wait()
        pack_cols(0, 1, 0)
        pltpu.sync_copy(
            obuf.at[pl.ds(0, 2048)], out_hbm.at[pl.ds(c * 2048, 2048)]
        )


@functools.partial(
    pl.kernel,
    out_type=jax.ShapeDtypeStruct((B, DIM), jnp.float32),
    mesh=_mesh2,
    scratch_types=[
        pltpu.VMEM((NG, IW), jnp.int32),
        pltpu.VMEM((ROWS, DIM // 2), jnp.float32),
        pltpu.VMEM((C, DIM), jnp.float32),
        pltpu.SemaphoreType.DMA,
    ],
    compiler_params=pltpu.CompilerParams(
        use_tc_tiling_on_sc=False, needs_layout_passes=False
    ),
)
def _embed_kernel(tw_hbm, batch_hbm, out_hbm, idx_v, rows_v, out_v, sem):
    wid = lax.axis_index("s") * NC + lax.axis_index("c")
    lane = lax.iota(jnp.int32, 16)
    even = lane * 2
    odd = even + 1

    def chunk_body(ci, _):
        idx_row0 = wid * (SPW * L // IW) + ci * NG
        pltpu.sync_copy(batch_hbm.at[pl.ds(idx_row0, NG)], idx_v)
        copies = [
            pltpu.async_copy(
                tw_hbm.at[idx_v.at[j]],
                rows_v.at[pl.ds(j * IW, IW)],
                sem,
            )
            for j in range(NG)
        ]
        for cp in copies:
            cp.wait()

        def acc_body(l, accs):
            out = []
            for s in range(C):
                ae, ao = accs[s]
                row = plsc.bitcast(rows_v[s * L + l, :], jnp.bfloat16)
                e, o = plsc.unpack(row, format=plsc.PackFormat.INTERLEAVED)
                out.append((ae + e, ao + o))
            return tuple(out)

        zero = jnp.zeros((16,), jnp.float32)
        init = tuple((zero, zero) for _ in range(C))
        accs = lax.fori_loop(0, L, acc_body, init)
        scale = jnp.float32(1.0 / L)
        for s in range(C):
            ae, ao = accs[s]
            srow = jnp.full((16,), s, jnp.int32)
            plsc.store_scatter(out_v, [srow, even], ae * scale)
            plsc.store_scatter(out_v, [srow, odd], ao * scale)

        base = wid * SPW + ci * C
        pltpu.sync_copy(out_v, out_hbm.at[pl.ds(base, C)])
        return 0

    lax.fori_loop(0, NCHUNK, chunk_body, 0)


def kernel(table, batch):
    words = _pack_kernel(table.T)
    tw = words.reshape(VPAD, DIM // 2)
    batch_r = batch.reshape(IDX_ROWS_TOTAL, IW)
    return _embed_kernel(tw, batch_r)


# speedup vs baseline: 1.6141x; 1.1161x over previous
import functools

import jax
import jax.numpy as jnp
from jax import lax
from jax.experimental import pallas as pl
from jax.experimental.pallas import tpu as pltpu
from jax.experimental.pallas import tpu_sc as plsc

VOCAB = 1000000
DIM = 32
B = 16384
L = 50

NC = 2
NS = 16
NW = NC * NS

TCOLS = (VOCAB + 127) // 128
VPAD = TCOLS * 128
WORDS = VPAD * (DIM // 2)
GCOLS = 4
NGRP = (TCOLS - 1) // GCOLS
GPW = (NGRP + NW - 1) // NW
GW = GCOLS * 128 * 16

SPW = B // NW
C = 16
ROWS = C * L
NCHUNK = SPW // C
IW = 100
NG = ROWS // IW
IDX_ROWS_TOTAL = B * L // IW

_mesh1 = plsc.VectorSubcoreMesh(core_axis_name="c", subcore_axis_name="s")
_mesh2 = plsc.VectorSubcoreMesh(core_axis_name="c", subcore_axis_name="s")


@functools.partial(
    pl.kernel,
    out_type=jax.ShapeDtypeStruct((WORDS,), jnp.float32),
    mesh=_mesh1,
    scratch_types=[
        pltpu.VMEM((2, 4, 8, GCOLS * 128), jnp.float32),
        pltpu.VMEM((2 * GW,), jnp.float32),
        pltpu.SemaphoreType.DMA,
        pltpu.SemaphoreType.DMA,
    ],
    compiler_params=pltpu.CompilerParams(
        use_tc_tiling_on_sc=True,
        needs_layout_passes=False,
        disable_bounds_checks=True,
    ),
)
def _pack_kernel(tt_hbm, out_hbm, ibuf, obuf, sem_in, sem_out):
    wid = lax.axis_index("s") * NC + lax.axis_index("c")
    lane16 = lax.iota(jnp.int32, 16) * 16
    g0 = wid * GPW
    ngrp_w = jnp.minimum(jnp.maximum(NGRP - g0, 0), GPW)

    def issue_in(g, pio):
        for db in range(4):
            pltpu.async_copy(
                tt_hbm.at[
                    pl.ds(8 * db, 8), pl.ds(g * (GCOLS * 128), GCOLS * 128)
                ],
                ibuf.at[pio, db],
                sem_in,
            )

    def pack_cols(pio, ncols, poff):
        for cc in range(ncols):
            for rc in range(8):
                rl0 = cc * 128 + rc * 16
                idx_base = lane16 + (rl0 * 16) + poff
                for j in range(16):
                    db = j // 4
                    ds = 2 * (j % 4)
                    e = ibuf[pio, db, ds, pl.ds(rl0, 16)]
                    o = ibuf[pio, db, ds + 1, pl.ds(rl0, 16)]
                    w = plsc.bitcast(
                        plsc.pack(e, o, format=plsc.PackFormat.INTERLEAVED),
                        jnp.float32,
                    )
                    plsc.store_scatter(obuf, [idx_base + j], w)

    @pl.when(ngrp_w >= 1)
    def _():
        issue_in(g0, 0)

    def grp_body(i, _):
        g = g0 + i

        @pl.when(g < NGRP)
        def _():
            pio = lax.rem(i, 2)
            poff = pio * GW
            for db in range(4):
                pltpu.make_async_copy(
                    tt_hbm.at[pl.ds(0, 8), pl.ds(0, GCOLS * 128)],
                    ibuf.at[0, db],
                    sem_in,
                ).wait()
            @pl.when(g + 1 < jnp.minimum(g0 + GPW, NGRP))
            def _():
                issue_in(g + 1, 1 - pio)

            @pl.when(i >= 2)
            def _():
                pltpu.make_async_copy(
                    obuf.at[pl.ds(0, GW)], out_hbm.at[pl.ds(0, GW)], sem_out
                ).wait()

            pack_cols(pio, GCOLS, poff)
            pltpu.async_copy(
                obuf.at[pl.ds(poff, GW)], out_hbm.at[pl.ds(g * GW, GW)], sem_out
            )

        return 0

    lax.fori_loop(0, GPW, grp_body, 0)

    @pl.when(ngrp_w >= 1)
    def _():
        pltpu.make_async_copy(
            obuf.at[pl.ds(0, GW)], out_hbm.at[pl.ds(0, GW)], sem_out
        ).wait()

    @pl.when(ngrp_w >= 2)
    def _():
        pltpu.make_async_copy(
            obuf.at[pl.ds(0, GW)], out_hbm.at[pl.ds(0, GW)], sem_out
        ).wait()

    @pl.when(wid == 0)
    def _():
        c = NGRP * GCOLS + wid
        cps = [
            pltpu.async_copy(
                tt_hbm.at[pl.ds(8 * db, 8), pl.ds(c * 128, 128)],
                ibuf.at[0, db, :, pl.ds(0, 128)],
                sem_in,
            )
            for db in range(4)
        ]
        for cp in cps:
            cp.wait()
        pack_cols(0, 1, 0)
        pltpu.sync_copy(
            obuf.at[pl.ds(0, 2048)], out_hbm.at[pl.ds(c * 2048, 2048)]
        )


@functools.partial(
    pl.kernel,
    out_type=jax.ShapeDtypeStruct((B, DIM), jnp.float32),
    mesh=_mesh2,
    scratch_types=[
        pltpu.VMEM((2, NG, IW), jnp.int32),
        pltpu.VMEM((2, ROWS, DIM // 2), jnp.float32),
        pltpu.VMEM((C, DIM), jnp.float32),
        pltpu.SemaphoreType.DMA,
        pltpu.SemaphoreType.DMA,
    ],
    compiler_params=pltpu.CompilerParams(
        use_tc_tiling_on_sc=False, needs_layout_passes=False
    ),
)
def _embed_kernel(tw_hbm, batch_hbm, out_hbm, idx_v, rows_v, out_v, sem0, sem1):
    wid = lax.axis_index("s") * NC + lax.axis_index("c")
    lane = lax.iota(jnp.int32, 16)
    even = lane * 2
    odd = even + 1
    idx_base0 = wid * (SPW * L // IW)

    def fetch_chunk(ci, p, sem):
        pltpu.sync_copy(
            batch_hbm.at[pl.ds(idx_base0 + ci * NG, NG)], idx_v.at[p]
        )
        for j in range(NG):
            pltpu.async_copy(
                tw_hbm.at[idx_v.at[p, j]],
                rows_v.at[p, pl.ds(j * IW, IW)],
                sem,
            )

    def drain_chunk(sem):
        for j in range(NG):
            pltpu.make_async_copy(
                tw_hbm.at[idx_v.at[0, 0]],
                rows_v.at[0, pl.ds(0, IW)],
                sem,
            ).wait()

    fetch_chunk(0, 0, sem0)

    def chunk_body(ci, _):
        p = lax.rem(ci, 2)

        @pl.when(jnp.logical_and(p == 0, ci + 1 < NCHUNK))
        def _():
            fetch_chunk(ci + 1, 1, sem1)

        @pl.when(jnp.logical_and(p == 1, ci + 1 < NCHUNK))
        def _():
            fetch_chunk(ci + 1, 0, sem0)

        @pl.when(p == 0)
        def _():
            drain_chunk(sem0)

        @pl.when(p == 1)
        def _():
            drain_chunk(sem1)

        def acc_body(l, accs):
            out = []
            for s in range(C):
                ae, ao = accs[s]
                row = plsc.bitcast(rows_v[p, s * L + l, :], jnp.bfloat16)
                e, o = plsc.unpack(row, format=plsc.PackFormat.INTERLEAVED)
                out.append((ae + e, ao + o))
            return tuple(out)

        zero = jnp.zeros((16,), jnp.float32)
        init = tuple((zero, zero) for _ in range(C))
        accs = lax.fori_loop(0, L, acc_body, init)
        scale = jnp.float32(1.0 / L)
        for s in range(C):
            ae, ao = accs[s]
            srow = jnp.full((16,), s, jnp.int32)
            plsc.store_scatter(out_v, [srow, even], ae * scale)
            plsc.store_scatter(out_v, [srow, odd], ao * scale)

        base = wid * SPW + ci * C
        pltpu.sync_copy(out_v, out_hbm.at[pl.ds(base, C)])
        return 0

    lax.fori_loop(0, NCHUNK, chunk_body, 0)


def kernel(table, batch):
    words = _pack_kernel(table.T)
    tw = words.reshape(VPAD, DIM // 2)
    batch_r = batch.reshape(IDX_ROWS_TOTAL, IW)
    return _embed_kernel(tw, batch_r)
